# Initial kernel scaffold; baseline (speedup 1.0000x reference)
import functools

import jax
import jax.numpy as jnp
from jax import lax
from jax.experimental import pallas as pl
from jax.experimental.pallas import tpu as pltpu
from jax.experimental.pallas import tpu_sc as plsc

DIM = 32
B = 16384 * 50

_info = plsc.get_sparse_core_info()
_NC, _NS = _info.num_cores, _info.num_subcores
NW = _NC * _NS
B_PER_W = B // NW
CHUNK = 1600
NCHUNK = B_PER_W // CHUNK

_mesh = plsc.VectorSubcoreMesh(core_axis_name="c", subcore_axis_name="s")


@functools.partial(
    pl.kernel,
    mesh=_mesh,
    out_type=jax.ShapeDtypeStruct((B, DIM), jnp.float32),
    scratch_types=[
        pltpu.VMEM((CHUNK,), jnp.int32),
        pltpu.VMEM((CHUNK, DIM), jnp.float32),
        pltpu.SemaphoreType.DMA,
    ],
)
def _gather_kernel(idx_hbm, table_hbm, out_hbm, idx_v, rows_v, sem):
    wid = lax.axis_index("s") * _NC + lax.axis_index("c")
    base = wid * B_PER_W

    def body(i, carry):
        off = base + i * CHUNK
        pltpu.sync_copy(idx_hbm.at[pl.ds(off, CHUNK)], idx_v)
        pltpu.async_copy(table_hbm.at[idx_v], rows_v, sem).wait()
        pltpu.sync_copy(rows_v, out_hbm.at[pl.ds(off, CHUNK)])
        return carry

    lax.fori_loop(0, NCHUNK, body, 0)


def kernel(substructure_indices, embedding_table):
    flat_idx = substructure_indices.reshape(-1).astype(jnp.int32)
    out = _gather_kernel(flat_idx, embedding_table)
    return out.reshape(substructure_indices.shape + (DIM,))


# speedup vs baseline: 1.1028x; 1.1028x over previous
import functools

import jax
import jax.numpy as jnp
from jax import lax
from jax.experimental import pallas as pl
from jax.experimental.pallas import tpu as pltpu
from jax.experimental.pallas import tpu_sc as plsc

DIM = 32
B = 16384 * 50

_info = plsc.get_sparse_core_info()
_NC, _NS = _info.num_cores, _info.num_subcores
NW = _NC * _NS
B_PER_W = B // NW
CHUNK = 1600
NCHUNK = B_PER_W // CHUNK

_mesh = plsc.VectorSubcoreMesh(core_axis_name="c", subcore_axis_name="s")


@functools.partial(
    pl.kernel,
    mesh=_mesh,
    out_type=jax.ShapeDtypeStruct((B, DIM), jnp.float32),
    scratch_types=[
        pltpu.VMEM((CHUNK,), jnp.int32),
        pltpu.VMEM((CHUNK, DIM), jnp.float32),
        pltpu.SemaphoreType.DMA,
    ],
    compiler_params=pltpu.CompilerParams(use_tc_tiling_on_sc=False),
)
def _gather_kernel(idx_hbm, table_hbm, out_hbm, idx_v, rows_v, sem):
    wid = lax.axis_index("s") * _NC + lax.axis_index("c")
    base = wid * B_PER_W

    def body(i, carry):
        off = base + i * CHUNK
        pltpu.sync_copy(idx_hbm.at[pl.ds(off, CHUNK)], idx_v)
        pltpu.async_copy(table_hbm.at[idx_v], rows_v, sem).wait()
        pltpu.sync_copy(rows_v, out_hbm.at[pl.ds(off, CHUNK)])
        return carry

    lax.fori_loop(0, NCHUNK, body, 0)


def kernel(substructure_indices, embedding_table):
    flat_idx = substructure_indices.reshape(-1).astype(jnp.int32)
    out = _gather_kernel(flat_idx, embedding_table)
    return out.reshape(substructure_indices.shape + (DIM,))


# speedup vs baseline: 1.1131x; 1.0093x over previous
import functools

import jax
import jax.numpy as jnp
from jax import lax
from jax.experimental import pallas as pl
from jax.experimental.pallas import tpu as pltpu
from jax.experimental.pallas import tpu_sc as plsc

DIM = 32
B = 16384 * 50

_info = plsc.get_sparse_core_info()
_NC, _NS = _info.num_cores, _info.num_subcores
NW = _NC * _NS
B_PER_W = B // NW
NBUF = 4
CHUNK = 800
NCHUNK = B_PER_W // CHUNK

_mesh = plsc.VectorSubcoreMesh(core_axis_name="c", subcore_axis_name="s")


@functools.partial(
    pl.kernel,
    mesh=_mesh,
    out_type=jax.ShapeDtypeStruct((B, DIM), jnp.float32),
    scratch_types=[
        pltpu.VMEM((B_PER_W,), jnp.int32),
        *[pltpu.VMEM((CHUNK, DIM), jnp.float32) for _ in range(NBUF)],
        *[pltpu.SemaphoreType.DMA for _ in range(2 * NBUF)],
    ],
    compiler_params=pltpu.CompilerParams(use_tc_tiling_on_sc=False),
)
def _gather_kernel(idx_hbm, table_hbm, out_hbm, idx_all, *bufs_and_sems):
    rows = bufs_and_sems[:NBUF]
    sem_g = bufs_and_sems[NBUF:2 * NBUF]
    sem_o = bufs_and_sems[2 * NBUF:]
    wid = lax.axis_index("s") * _NC + lax.axis_index("c")
    base = wid * B_PER_W

    pltpu.sync_copy(idx_hbm.at[pl.ds(base, B_PER_W)], idx_all)

    gd = [None] * NCHUNK
    sd = [None] * NCHUNK

    def fire_gather(i):
        b = i % NBUF
        gd[i] = pltpu.async_copy(
            table_hbm.at[idx_all.at[pl.ds(i * CHUNK, CHUNK)]], rows[b], sem_g[b])

    for i in range(NBUF):
        fire_gather(i)

    for i in range(NCHUNK):
        b = i % NBUF
        gd[i].wait()
        sd[i] = pltpu.async_copy(
            rows[b], out_hbm.at[pl.ds(base + i * CHUNK, CHUNK)], sem_o[b])
        ni = i + NBUF - 1
        if i >= 1 and ni < NCHUNK:
            sd[i - 1].wait()
            fire_gather(ni)

    for i in range(NCHUNK - NBUF, NCHUNK):
        sd[i].wait()


def kernel(substructure_indices, embedding_table):
    flat_idx = substructure_indices.reshape(-1).astype(jnp.int32)
    out = _gather_kernel(flat_idx, embedding_table)
    return out.reshape(substructure_indices.shape + (DIM,))


# speedup vs baseline: 1.8015x; 1.6184x over previous
import functools

import jax
import jax.numpy as jnp
from jax import lax
from jax.experimental import pallas as pl
from jax.experimental.pallas import tpu as pltpu
from jax.experimental.pallas import tpu_sc as plsc

DIM = 32
NROW = 16384
SEQ = 50

_info = plsc.get_sparse_core_info()
_NC, _NS = _info.num_cores, _info.num_subcores
NW = _NC * _NS
ROWS_PER_W = NROW // NW
NBUF = 4
CROWS = 16
NCHUNK = ROWS_PER_W // CROWS

_mesh = plsc.VectorSubcoreMesh(core_axis_name="c", subcore_axis_name="s")


@functools.partial(
    pl.kernel,
    mesh=_mesh,
    out_type=jax.ShapeDtypeStruct((NROW, SEQ, DIM), jnp.float32),
    scratch_types=[
        pltpu.VMEM((ROWS_PER_W, SEQ), jnp.int32),
        *[pltpu.VMEM((CROWS, SEQ, DIM), jnp.float32) for _ in range(NBUF)],
        *[pltpu.SemaphoreType.DMA for _ in range(2 * NBUF)],
    ],
    compiler_params=pltpu.CompilerParams(use_tc_tiling_on_sc=False),
)
def _gather_kernel(idx_hbm, table_hbm, out_hbm, idx_v, *bufs_and_sems):
    rows = bufs_and_sems[:NBUF]
    sem_g = bufs_and_sems[NBUF:2 * NBUF]
    sem_o = bufs_and_sems[2 * NBUF:]
    wid = lax.axis_index("s") * _NC + lax.axis_index("c")
    base_row = wid * ROWS_PER_W

    pltpu.sync_copy(idx_hbm.at[pl.ds(base_row, ROWS_PER_W)], idx_v)

    def fire_gathers(i, b):
        def body(k, carry):
            pltpu.make_async_copy(
                table_hbm.at[idx_v.at[i * CROWS + k]], rows[b].at[k], sem_g[b]
            ).start()
            return carry
        lax.fori_loop(0, CROWS, body, 0)

    def wait_gathers(b):
        def body(k, carry):
            pltpu.make_async_copy(
                table_hbm.at[idx_v.at[k]], rows[b].at[k], sem_g[b]).wait()
            return carry
        lax.fori_loop(0, CROWS, body, 0)

    sd = [None] * NCHUNK

    for i in range(NBUF):
        fire_gathers(i, i)

    for i in range(NCHUNK):
        b = i % NBUF
        wait_gathers(b)
        sd[i] = pltpu.async_copy(
            rows[b], out_hbm.at[pl.ds(base_row + i * CROWS, CROWS)], sem_o[b])
        ni = i + NBUF - 1
        if i >= 1 and ni < NCHUNK:
            sd[i - 1].wait()
            fire_gathers(ni, (i - 1) % NBUF)

    for i in range(NCHUNK - NBUF, NCHUNK):
        sd[i].wait()


def kernel(substructure_indices, embedding_table):
    return _gather_kernel(substructure_indices.astype(jnp.int32), embedding_table)
